# baseline (speedup 1.0000x reference)
import functools

import numpy as np
import jax
import jax.numpy as jnp
from jax import lax
from jax.experimental import pallas as pl
from jax.experimental.pallas import tpu as pltpu
from jax.experimental.pallas import tpu_sc as plsc

VOCAB = 100000
D = 64
B = 4096
L = 200
N = B * L

NC = 2
NS = 16
NW = NC * NS

BBLK = 128
NBT = B // BBLK
NTASK = L * NBT
TASKS_PER_W = NTASK // NW
NBUF = 4
LW = 8
OPAD = 133


def _positional_encoding() -> np.ndarray:
    pos = np.arange(L, dtype=np.float64)[:, None]
    idx = np.arange(D, dtype=np.float64)[None, :]
    inner = pos / np.power(10000.0, 2.0 * idx / D)
    even = (np.arange(D)[None, :] % 2) == 0
    pe = np.where(even, np.sin(inner), np.cos(inner))
    return pe.astype(np.float32)


_PE8 = np.zeros((L + LW, D), dtype=np.float32)
_PE8[:L] = _positional_encoding() * 8.0


def _sc_embed(xt_flat, table, pe8):
    mesh = plsc.VectorSubcoreMesh(
        core_axis_name="c", subcore_axis_name="s", num_cores=NC, num_subcores=NS
    )

    @functools.partial(
        pl.kernel,
        out_type=jax.ShapeDtypeStruct((L, D // 8, NBT, 8, BBLK), jnp.float32),
        mesh=mesh,
        scratch_types=[
            pltpu.VMEM((LW, D), jnp.float32),
            [pltpu.VMEM((BBLK,), jnp.int32) for _ in range(NBUF)],
            [pltpu.VMEM((BBLK, D), jnp.float32) for _ in range(NBUF)],
            [pltpu.VMEM((D // 8, 8, OPAD), jnp.float32) for _ in range(NBUF)],
            [pltpu.SemaphoreType.DMA for _ in range(NBUF)],
            [pltpu.SemaphoreType.DMA for _ in range(NBUF)],
        ],
        compiler_params=pltpu.CompilerParams(
            use_tc_tiling_on_sc=False, needs_layout_passes=False
        ),
    )
    def k(xt_hbm, tab_hbm, pe8_hbm, out_hbm, pe8_v, idx_v, g_v, o_v, gsem, wsem):
        wid = lax.axis_index("s") * NC + lax.axis_index("c")
        base = wid * TASKS_PER_W
        lmin = base // NBT

        pltpu.sync_copy(pe8_hbm.at[pl.ds(lmin, LW)], pe8_v)
        iota = lax.iota(jnp.int32, 16)
        dt_base = iota // 8
        di_vec = lax.rem(iota, 8)

        def start_gather(b, ci):
            tid = base + ci
            lpos = tid // NBT
            bt = tid % NBT
            pltpu.sync_copy(xt_hbm.at[pl.ds(lpos * B + bt * BBLK, BBLK)], idx_v[b])
            pltpu.async_copy(tab_hbm.at[idx_v[b]], g_v[b], gsem[b])

        for b in range(NBUF - 1):
            start_gather(b, b)

        def compute(b, lpos):
            g = g_v[b]
            o = o_v[b]
            lrel = lpos - lmin

            for kk in range(D // 16):
                pe8vec = pe8_v[lrel, pl.ds(kk * 16, 16)]
                dt_vec = dt_base + (2 * kk)

                @plsc.parallel_loop(0, BBLK, unroll=4)
                def body(bb):
                    v = g[bb, pl.ds(kk * 16, 16)]
                    res = v * 8.0 + pe8vec
                    b_vec = jnp.full((16,), bb, dtype=jnp.int32)
                    plsc.store_scatter(o, [dt_vec, di_vec, b_vec], res)

        def out_slice(ci):
            tid = base + ci
            return out_hbm.at[tid // NBT, :, tid % NBT]

        def o_src(b):
            return o_v[b].at[:, :, pl.ds(0, BBLK)]

        def step(it, carry):
            for b in range(NBUF):
                ci = it * NBUF + b
                tid = base + ci
                pltpu.make_async_copy(tab_hbm.at[idx_v[b]], g_v[b], gsem[b]).wait()
                compute(b, tid // NBT)
                pltpu.async_copy(o_src(b), out_slice(ci), wsem[b])

                nci = ci + NBUF - 1
                pb = (b + NBUF - 1) % NBUF

                @pl.when(nci < TASKS_PER_W)
                def _prep():
                    @pl.when(ci >= 1)
                    def _drain_prev_write():
                        pltpu.make_async_copy(o_src(pb), out_slice(ci - 1), wsem[pb]).wait()

                    start_gather(pb, nci)

            return carry

        lax.fori_loop(0, TASKS_PER_W // NBUF, step, 0)

        for b in range(NBUF):
            ci = TASKS_PER_W - NBUF + b
            pltpu.make_async_copy(o_src(b), out_slice(ci), wsem[b]).wait()

    return k(xt_flat, table, pe8)


def kernel(x, table):
    pe8 = jnp.asarray(_PE8)
    xt = x.T.reshape(N)
    out5 = _sc_embed(xt, table, pe8)
    return out5.transpose(2, 4, 0, 1, 3).reshape(B, L, D)


# speedup vs baseline: 1.4509x; 1.4509x over previous
import functools

import numpy as np
import jax
import jax.numpy as jnp
from jax import lax
from jax.experimental import pallas as pl
from jax.experimental.pallas import tpu as pltpu
from jax.experimental.pallas import tpu_sc as plsc

VOCAB = 100000
D = 64
B = 4096
L = 200
N = B * L

NC = 2
NS = 16
NW = NC * NS

BBLK = 128
NBT = B // BBLK
NTASK = L * NBT
TASKS_PER_W = NTASK // NW
NBUF = 4
LW = 8
OPAD = 133


def _positional_encoding() -> np.ndarray:
    pos = np.arange(L, dtype=np.float64)[:, None]
    idx = np.arange(D, dtype=np.float64)[None, :]
    inner = pos / np.power(10000.0, 2.0 * idx / D)
    even = (np.arange(D)[None, :] % 2) == 0
    pe = np.where(even, np.sin(inner), np.cos(inner))
    return pe.astype(np.float32)


_PE8 = np.zeros((L + LW, D), dtype=np.float32)
_PE8[:L] = _positional_encoding() * 8.0


def _sc_embed(xt_flat, table, pe8):
    mesh = plsc.VectorSubcoreMesh(
        core_axis_name="c", subcore_axis_name="s", num_cores=NC, num_subcores=NS
    )

    @functools.partial(
        pl.kernel,
        out_type=jax.ShapeDtypeStruct((L, D // 8, NBT, 8, BBLK), jnp.float32),
        mesh=mesh,
        scratch_types=[
            pltpu.VMEM((LW, D), jnp.float32),
            pltpu.VMEM((TASKS_PER_W * BBLK,), jnp.int32),
            [pltpu.VMEM((BBLK, D), jnp.float32) for _ in range(NBUF)],
            [pltpu.VMEM((D // 8, 8, OPAD), jnp.float32) for _ in range(NBUF)],
            [pltpu.SemaphoreType.DMA for _ in range(NBUF)],
            [pltpu.SemaphoreType.DMA for _ in range(NBUF)],
        ],
        compiler_params=pltpu.CompilerParams(
            use_tc_tiling_on_sc=False, needs_layout_passes=False
        ),
    )
    def k(xt_hbm, tab_hbm, pe8_hbm, out_hbm, pe8_v, idx_v, g_v, o_v, gsem, wsem):
        wid = lax.axis_index("s") * NC + lax.axis_index("c")
        base = wid * TASKS_PER_W
        lmin = base // NBT

        pltpu.sync_copy(pe8_hbm.at[pl.ds(lmin, LW)], pe8_v)
        pltpu.sync_copy(xt_hbm.at[pl.ds(base * BBLK, TASKS_PER_W * BBLK)], idx_v)
        iota = lax.iota(jnp.int32, 16)
        dt_base = iota // 8
        di_vec = lax.rem(iota, 8)

        def idx_slice(ci):
            return idx_v.at[pl.ds(ci * BBLK, BBLK)]

        def start_gather(b, ci):
            pltpu.async_copy(tab_hbm.at[idx_slice(ci)], g_v[b], gsem[b])

        for b in range(NBUF - 1):
            start_gather(b, b)

        def compute(b, lpos):
            g = g_v[b]
            o = o_v[b]
            lrel = lpos - lmin

            for kk in range(D // 16):
                pe8vec = pe8_v[lrel, pl.ds(kk * 16, 16)]
                dt_vec = dt_base + (2 * kk)

                @plsc.parallel_loop(0, BBLK, unroll=4)
                def body(bb):
                    v = g[bb, pl.ds(kk * 16, 16)]
                    res = v * 8.0 + pe8vec
                    b_vec = jnp.full((16,), bb, dtype=jnp.int32)
                    plsc.store_scatter(o, [dt_vec, di_vec, b_vec], res)

        def out_slice(ci):
            tid = base + ci
            return out_hbm.at[tid // NBT, :, tid % NBT]

        def o_src(b):
            return o_v[b].at[:, :, pl.ds(0, BBLK)]

        def step(it, carry):
            for b in range(NBUF):
                ci = it * NBUF + b
                tid = base + ci
                pltpu.make_async_copy(tab_hbm.at[idx_slice(ci)], g_v[b], gsem[b]).wait()
                compute(b, tid // NBT)
                pltpu.async_copy(o_src(b), out_slice(ci), wsem[b])

                nci = ci + NBUF - 1
                pb = (b + NBUF - 1) % NBUF

                @pl.when(nci < TASKS_PER_W)
                def _prep():
                    @pl.when(ci >= 1)
                    def _drain_prev_write():
                        pltpu.make_async_copy(o_src(pb), out_slice(ci - 1), wsem[pb]).wait()

                    start_gather(pb, nci)

            return carry

        lax.fori_loop(0, TASKS_PER_W // NBUF, step, 0)

        for b in range(NBUF):
            ci = TASKS_PER_W - NBUF + b
            pltpu.make_async_copy(o_src(b), out_slice(ci), wsem[b]).wait()

    return k(xt_flat, table, pe8)


def kernel(x, table):
    pe8 = jnp.asarray(_PE8)
    xt = x.T.reshape(N)
    out5 = _sc_embed(xt, table, pe8)
    return out5.transpose(2, 4, 0, 1, 3).reshape(B, L, D)


# speedup vs baseline: 1.5081x; 1.0395x over previous
import functools

import numpy as np
import jax
import jax.numpy as jnp
from jax import lax
from jax.experimental import pallas as pl
from jax.experimental.pallas import tpu as pltpu
from jax.experimental.pallas import tpu_sc as plsc

VOCAB = 100000
D = 64
B = 4096
L = 200
N = B * L

NC = 2
NS = 16
NW = NC * NS

BBLK = 128
NBT = B // BBLK
NTASK = L * NBT
TASKS_PER_W = NTASK // NW
NBUF = 5
LW = 8
OPAD = 133


def _positional_encoding() -> np.ndarray:
    pos = np.arange(L, dtype=np.float64)[:, None]
    idx = np.arange(D, dtype=np.float64)[None, :]
    inner = pos / np.power(10000.0, 2.0 * idx / D)
    even = (np.arange(D)[None, :] % 2) == 0
    pe = np.where(even, np.sin(inner), np.cos(inner))
    return pe.astype(np.float32)


_PE8 = np.zeros((L + LW, D), dtype=np.float32)
_PE8[:L] = _positional_encoding() * 8.0


def _sc_embed(xt_flat, table, pe8):
    mesh = plsc.VectorSubcoreMesh(
        core_axis_name="c", subcore_axis_name="s", num_cores=NC, num_subcores=NS
    )

    @functools.partial(
        pl.kernel,
        out_type=jax.ShapeDtypeStruct((L, D // 8, NBT, 8, BBLK), jnp.float32),
        mesh=mesh,
        scratch_types=[
            pltpu.VMEM((LW, D), jnp.float32),
            pltpu.VMEM((TASKS_PER_W * BBLK,), jnp.int32),
            [pltpu.VMEM((BBLK, D), jnp.float32) for _ in range(NBUF)],
            [pltpu.VMEM((D // 8, 8, OPAD), jnp.float32) for _ in range(NBUF)],
            [pltpu.SemaphoreType.DMA for _ in range(NBUF)],
            [pltpu.SemaphoreType.DMA for _ in range(NBUF)],
        ],
        compiler_params=pltpu.CompilerParams(
            use_tc_tiling_on_sc=False, needs_layout_passes=False
        ),
    )
    def k(xt_hbm, tab_hbm, pe8_hbm, out_hbm, pe8_v, idx_v, g_v, o_v, gsem, wsem):
        wid = lax.axis_index("s") * NC + lax.axis_index("c")
        base = wid * TASKS_PER_W
        lmin = base // NBT

        pltpu.sync_copy(pe8_hbm.at[pl.ds(lmin, LW)], pe8_v)
        pltpu.sync_copy(xt_hbm.at[pl.ds(base * BBLK, TASKS_PER_W * BBLK)], idx_v)
        iota = lax.iota(jnp.int32, 16)
        dt_base = iota // 8
        di_vec = lax.rem(iota, 8)

        def idx_slice(ci):
            return idx_v.at[pl.ds(ci * BBLK, BBLK)]

        def start_gather(b, ci):
            pltpu.async_copy(tab_hbm.at[idx_slice(ci)], g_v[b], gsem[b])

        for b in range(NBUF - 1):
            start_gather(b, b)

        def compute(b, lpos):
            g = g_v[b]
            o = o_v[b]
            lrel = lpos - lmin

            for kk in range(D // 16):
                pe8vec = pe8_v[lrel, pl.ds(kk * 16, 16)]
                dt_vec = dt_base + (2 * kk)

                @plsc.parallel_loop(0, BBLK, unroll=4)
                def body(bb):
                    v = g[bb, pl.ds(kk * 16, 16)]
                    res = v * 8.0 + pe8vec
                    b_vec = jnp.full((16,), bb, dtype=jnp.int32)
                    plsc.store_scatter(o, [dt_vec, di_vec, b_vec], res)

        def out_slice(ci):
            tid = base + ci
            return out_hbm.at[tid // NBT, :, tid % NBT]

        def o_src(b):
            return o_v[b].at[:, :, pl.ds(0, BBLK)]

        def step(it, carry):
            for b in range(NBUF):
                ci = it * NBUF + b
                tid = base + ci
                pltpu.make_async_copy(tab_hbm.at[idx_slice(ci)], g_v[b], gsem[b]).wait()
                compute(b, tid // NBT)
                pltpu.async_copy(o_src(b), out_slice(ci), wsem[b])

                nci = ci + NBUF - 1
                pb = (b + NBUF - 1) % NBUF

                @pl.when(nci < TASKS_PER_W)
                def _prep():
                    @pl.when(ci >= 1)
                    def _drain_prev_write():
                        pltpu.make_async_copy(o_src(pb), out_slice(ci - 1), wsem[pb]).wait()

                    start_gather(pb, nci)

            return carry

        lax.fori_loop(0, TASKS_PER_W // NBUF, step, 0)

        for b in range(NBUF):
            ci = TASKS_PER_W - NBUF + b
            pltpu.make_async_copy(o_src(b), out_slice(ci), wsem[b]).wait()

    return k(xt_flat, table, pe8)


def kernel(x, table):
    pe8 = jnp.asarray(_PE8)
    xt = x.T.reshape(N)
    out5 = _sc_embed(xt, table, pe8)
    return out5.transpose(2, 4, 0, 1, 3).reshape(B, L, D)
